# Initial kernel scaffold; baseline (speedup 1.0000x reference)
import jax
import jax.numpy as jnp
from jax.experimental import pallas as pl


def kernel(x, edge_index, W1, b1, W2, b2, W3, b3):
    raise NotImplementedError("write your pallas kernel here")


# speedup vs baseline: 1.4802x; 1.4802x over previous
import functools

import jax
import jax.numpy as jnp
from jax import lax
from jax.experimental import pallas as pl
from jax.experimental.pallas import tpu as pltpu

N = 10000
E = 320000


def _mm_body(x_ref, w_ref, o_ref):
    o_ref[...] = jnp.dot(x_ref[...], w_ref[...],
                         preferred_element_type=jnp.float32)


def _matmul(x, w, block_rows=2000):
    n, k = x.shape
    ko, m = w.shape
    grid = (n // block_rows,)
    return pl.pallas_call(
        _mm_body,
        grid=grid,
        in_specs=[
            pl.BlockSpec((block_rows, k), lambda i: (i, 0)),
            pl.BlockSpec((ko, m), lambda i: (0, 0)),
        ],
        out_specs=pl.BlockSpec((block_rows, m), lambda i: (i, 0)),
        out_shape=jax.ShapeDtypeStruct((n, m), jnp.float32),
    )(x, w)


def _relu_mm_body(a_ref, b_ref, w_ref, o_ref):
    h = jnp.maximum(a_ref[...] + b_ref[...], 0.0)
    o_ref[...] = jnp.dot(h, w_ref[...], preferred_element_type=jnp.float32)


def _relu_matmul(a, b, w, block_rows=2000):
    n, k = a.shape
    ko, m = w.shape
    b2 = b.reshape(1, k)
    grid = (n // block_rows,)
    return pl.pallas_call(
        _relu_mm_body,
        grid=grid,
        in_specs=[
            pl.BlockSpec((block_rows, k), lambda i: (i, 0)),
            pl.BlockSpec((1, k), lambda i: (0, 0)),
            pl.BlockSpec((ko, m), lambda i: (0, 0)),
        ],
        out_specs=pl.BlockSpec((block_rows, m), lambda i: (i, 0)),
        out_shape=jax.ShapeDtypeStruct((n, m), jnp.float32),
    )(a, b2, w)


def _bias_body(a_ref, b_ref, o_ref):
    o_ref[...] = a_ref[...] + b_ref[...]


def _bias_add(a, b, block_rows=2000):
    n, m = a.shape
    b2 = b.reshape(1, m)
    grid = (n // block_rows,)
    return pl.pallas_call(
        _bias_body,
        grid=grid,
        in_specs=[
            pl.BlockSpec((block_rows, m), lambda i: (i, 0)),
            pl.BlockSpec((1, m), lambda i: (0, 0)),
        ],
        out_specs=pl.BlockSpec((block_rows, m), lambda i: (i, 0)),
        out_shape=jax.ShapeDtypeStruct((n, m), jnp.float32),
    )(a, b2)


def _dists_body(er_ref, ec_ref, o_ref):
    i = pl.program_id(0)
    j = pl.program_id(1)
    acc = lax.dot_general(er_ref[...], ec_ref[...],
                          (((1,), (1,)), ((), ())),
                          preferred_element_type=jnp.float32)

    @pl.when(i == j)
    def _():
        ri = lax.broadcasted_iota(jnp.int32, acc.shape, 0)
        ci = lax.broadcasted_iota(jnp.int32, acc.shape, 1)
        o_ref[...] = jnp.where(ri == ci, 0.0, acc)

    @pl.when(i != j)
    def _():
        o_ref[...] = acc


def _dists(embeds, block=1024):
    n, d = embeds.shape
    grid = (pl.cdiv(n, block), pl.cdiv(n, block))
    return pl.pallas_call(
        _dists_body,
        grid=grid,
        in_specs=[
            pl.BlockSpec((block, d), lambda i, j: (i, 0)),
            pl.BlockSpec((block, d), lambda i, j: (j, 0)),
        ],
        out_specs=pl.BlockSpec((block, block), lambda i, j: (i, j)),
        out_shape=jax.ShapeDtypeStruct((n, n), jnp.float32),
    )(embeds, embeds)


def _aggregate(support, src, dst):
    return jax.ops.segment_sum(support[src], dst, num_segments=N)


def kernel(x, edge_index, W1, b1, W2, b2, W3, b3):
    src = edge_index[0]
    dst = edge_index[1]

    s1 = _matmul(x, W1)
    a1 = _aggregate(s1, src, dst)
    s2 = _relu_matmul(a1, b1, W2)
    a2 = _aggregate(s2, src, dst)
    s3 = _relu_matmul(a2, b2, W3)
    a3 = _aggregate(s3, src, dst)
    embeds = _bias_add(a3, b3)
    dists = _dists(embeds)
    return (embeds, dists)


# speedup vs baseline: 9.0402x; 6.1072x over previous
import functools

import jax
import jax.numpy as jnp
from jax import lax
from jax.experimental import pallas as pl
from jax.experimental.pallas import tpu as pltpu
from jax.experimental.pallas import tpu_sc as plsc

N = 10000
E = 320000

NC = 2
NS = 16
NW = NC * NS
EPW = E // NW
K = 80
CH = -(-EPW // K)
EPW_PAD = CH * K
N_PAD = 10240
RPS = N_PAD // NS
ZR = 32
NB = 4


def _make_sc_agg(D):
    mesh = plsc.VectorSubcoreMesh(core_axis_name="c", subcore_axis_name="s")

    @functools.partial(
        pl.kernel,
        out_type=jax.ShapeDtypeStruct((NC, N_PAD, D), jnp.float32),
        mesh=mesh,
        scratch_types=(
            [pltpu.VMEM((K,), jnp.int32) for _ in range(NB)]
            + [pltpu.VMEM((K,), jnp.int32) for _ in range(NB)]
            + [pltpu.VMEM((K, D), jnp.float32) for _ in range(NB)]
            + [
                pltpu.VMEM((ZR, D), jnp.float32),
                pltpu.VMEM_SHARED((N_PAD, D), jnp.float32),
            ]
            + [pltpu.SemaphoreType.DMA for _ in range(3 * NB + 1)]
        ),
    )
    def agg(src_hbm, dst_hbm, sup_hbm, out_hbm, *scr):
        sidx = scr[0:NB]
        didx = scr[NB:2 * NB]
        rb = scr[2 * NB:3 * NB]
        zbuf = scr[3 * NB]
        acc = scr[3 * NB + 1]
        semi = scr[3 * NB + 2:4 * NB + 2]
        semg = scr[4 * NB + 2:5 * NB + 2]
        sems = scr[5 * NB + 2:6 * NB + 2]
        semz = scr[6 * NB + 2]
        c = lax.axis_index("c")
        s = lax.axis_index("s")
        w = s * NC + c

        def istart(i, p):
            i = pl.multiple_of(i + 0 * w, 1)
            pltpu.async_copy(src_hbm.at[w, i], sidx[p], semi[p])
            pltpu.async_copy(dst_hbm.at[w, i], didx[p], semi[p])

        def iwait(i, p):
            i = pl.multiple_of(i + 0 * w, 1)
            pltpu.make_async_copy(src_hbm.at[w, i], sidx[p], semi[p]).wait()
            pltpu.make_async_copy(dst_hbm.at[w, i], didx[p], semi[p]).wait()

        def gstart(p):
            pltpu.async_copy(sup_hbm.at[sidx[p]], rb[p], semg[p])

        def gwait(p):
            pltpu.make_async_copy(sup_hbm.at[sidx[p]], rb[p], semg[p]).wait()

        def sstart(p):
            pltpu.async_copy(rb[p], acc.at[didx[p]], sems[p], add=True)

        def swait(p):
            pltpu.make_async_copy(rb[p], acc.at[didx[p]], sems[p]).wait()

        istart(0, 0)
        istart(1, 1)

        zero16 = jnp.zeros((16,), jnp.float32)

        def zrow(r, carry):
            for q in range(D // 16):
                zbuf[r, pl.ds(q * 16, 16)] = zero16
            return carry

        lax.fori_loop(0, ZR, zrow, 0)
        iwait(0, 0)
        gstart(0)
        istart(2, 2)
        for t in range(RPS // ZR):
            pltpu.async_copy(zbuf, acc.at[pl.ds(s * RPS + t * ZR, ZR)], semz)
        iwait(1, 1)
        gstart(1)
        istart(3, 3)
        for t in range(RPS // ZR):
            pltpu.make_async_copy(zbuf, acc.at[pl.ds(s * RPS + t * ZR, ZR)],
                                  semz).wait()
        plsc.subcore_barrier()
        gwait(0)
        sstart(0)

        def slot(m, u):
            p = (2 + u) % NB
            pm1 = (1 + u) % NB
            pm2 = u % NB
            gwait(pm1)
            sstart(pm1)
            swait(pm2)
            istart(m + 2, pm2)
            iwait(m, p)
            gstart(p)

        def quad(j, carry):
            m = 2 + 4 * j
            for u in range(4):
                slot(m + u, u)
            return carry

        n_quads = (CH - 5) // 4
        lax.fori_loop(0, n_quads, quad, 0)
        for u in range(3):
            slot(CH - 3 + u, u)

        gwait((CH - 1) % NB)
        sstart((CH - 1) % NB)
        swait((CH - 2) % NB)
        swait((CH - 1) % NB)
        iwait(CH, CH % NB)
        iwait(CH + 1, (CH + 1) % NB)

        plsc.subcore_barrier()
        pltpu.sync_copy(acc.at[pl.ds(s * RPS, RPS)],
                        out_hbm.at[c, pl.ds(s * RPS, RPS)])

    return agg


_agg_cache = {}


def _aggregate(support, src3, dst3):
    D = support.shape[1]
    if D not in _agg_cache:
        _agg_cache[D] = _make_sc_agg(D)
    return _agg_cache[D](src3, dst3, support)


def _mm_body(x_ref, w_ref, o_ref):
    o_ref[...] = jnp.dot(x_ref[...], w_ref[...],
                         preferred_element_type=jnp.float32)


def _matmul(x, w, block_rows=2000):
    n, k = x.shape
    ko, m = w.shape
    grid = (n // block_rows,)
    return pl.pallas_call(
        _mm_body,
        grid=grid,
        in_specs=[
            pl.BlockSpec((block_rows, k), lambda i: (i, 0)),
            pl.BlockSpec((ko, m), lambda i: (0, 0)),
        ],
        out_specs=pl.BlockSpec((block_rows, m), lambda i: (i, 0)),
        out_shape=jax.ShapeDtypeStruct((n, m), jnp.float32),
    )(x, w)


def _relu_mm_body(p_ref, b_ref, w_ref, o_ref):
    h = jnp.maximum(p_ref[0] + p_ref[1] + b_ref[...], 0.0)
    o_ref[...] = jnp.dot(h, w_ref[...], preferred_element_type=jnp.float32)


def _relu_matmul(p, b, w, block_rows=2000):
    _, _, k = p.shape
    n = N
    ko, m = w.shape
    b2 = b.reshape(1, k)
    grid = (n // block_rows,)
    return pl.pallas_call(
        _relu_mm_body,
        grid=grid,
        in_specs=[
            pl.BlockSpec((NC, block_rows, k), lambda i: (0, i, 0)),
            pl.BlockSpec((1, k), lambda i: (0, 0)),
            pl.BlockSpec((ko, m), lambda i: (0, 0)),
        ],
        out_specs=pl.BlockSpec((block_rows, m), lambda i: (i, 0)),
        out_shape=jax.ShapeDtypeStruct((n, m), jnp.float32),
    )(p, b2, w)


def _bias_body(p_ref, b_ref, o_ref):
    m = o_ref.shape[-1]
    o_ref[...] = (p_ref[0] + p_ref[1] + b_ref[...])[:, :m]


def _bias_add(p, b, m_out, block_rows=2000):
    _, _, m = p.shape
    n = N
    b2 = b.reshape(1, m)
    grid = (n // block_rows,)
    return pl.pallas_call(
        _bias_body,
        grid=grid,
        in_specs=[
            pl.BlockSpec((NC, block_rows, m), lambda i: (0, i, 0)),
            pl.BlockSpec((1, m), lambda i: (0, 0)),
        ],
        out_specs=pl.BlockSpec((block_rows, m_out), lambda i: (i, 0)),
        out_shape=jax.ShapeDtypeStruct((n, m_out), jnp.float32),
    )(p, b2)


def _dists_body(er_ref, ec_ref, o_ref):
    i = pl.program_id(0)
    j = pl.program_id(1)
    acc = lax.dot_general(er_ref[...], ec_ref[...],
                          (((1,), (1,)), ((), ())),
                          preferred_element_type=jnp.float32)

    @pl.when(i == j)
    def _():
        ri = lax.broadcasted_iota(jnp.int32, acc.shape, 0)
        ci = lax.broadcasted_iota(jnp.int32, acc.shape, 1)
        o_ref[...] = jnp.where(ri == ci, 0.0, acc)

    @pl.when(i != j)
    def _():
        o_ref[...] = acc


def _dists(embeds, block=1024):
    n, d = embeds.shape
    grid = (pl.cdiv(n, block), pl.cdiv(n, block))
    return pl.pallas_call(
        _dists_body,
        grid=grid,
        in_specs=[
            pl.BlockSpec((block, d), lambda i, j: (i, 0)),
            pl.BlockSpec((block, d), lambda i, j: (j, 0)),
        ],
        out_specs=pl.BlockSpec((block, block), lambda i, j: (i, j)),
        out_shape=jax.ShapeDtypeStruct((n, n), jnp.float32),
    )(embeds, embeds)


def kernel(x, edge_index, W1, b1, W2, b2, W3, b3):
    e2 = edge_index.reshape(2, NW, EPW)
    src_p = jnp.pad(e2[0], ((0, 0), (0, EPW_PAD - EPW)))
    pad_dst = jnp.broadcast_to(N + jnp.arange(NW, dtype=jnp.int32)[:, None],
                               (NW, EPW_PAD - EPW)) if EPW_PAD > EPW else None
    dst_p = (jnp.concatenate([e2[1], pad_dst], axis=1)
             if EPW_PAD > EPW else e2[1])
    pad = jnp.zeros((NW, 2, K), jnp.int32)
    src3 = jnp.concatenate([src_p.reshape(NW, CH, K), pad], axis=1)
    dst3 = jnp.concatenate([dst_p.reshape(NW, CH, K), pad], axis=1)

    W3p = jnp.pad(W3, ((0, 0), (0, 128 - W3.shape[1])))
    b3p = jnp.pad(b3, (0, 128 - b3.shape[0]))

    s1 = _matmul(x, W1)
    p1 = _aggregate(s1, src3, dst3)
    s2 = _relu_matmul(p1, b1, W2)
    p2 = _aggregate(s2, src3, dst3)
    s3 = _relu_matmul(p2, b2, W3p)
    p3 = _aggregate(s3, src3, dst3)
    embeds = _bias_add(p3, b3p, W3.shape[1])
    dists = _dists(embeds)
    return (embeds, dists)


# speedup vs baseline: 9.3745x; 1.0370x over previous
import functools

import jax
import jax.numpy as jnp
from jax import lax
from jax.experimental import pallas as pl
from jax.experimental.pallas import tpu as pltpu
from jax.experimental.pallas import tpu_sc as plsc

N = 10000
E = 320000

NC = 2
NS = 16
NW = NC * NS
EPW = E // NW
K = 80
CH = -(-EPW // K)
EPW_PAD = CH * K
N_PAD = 10240
RPS = N_PAD // NS
ZR = 64


def _make_sc_agg(D):
    mesh = plsc.VectorSubcoreMesh(core_axis_name="c", subcore_axis_name="s")

    @functools.partial(
        pl.kernel,
        out_type=jax.ShapeDtypeStruct((NC, N_PAD, D), jnp.float32),
        mesh=mesh,
        scratch_types=[
            pltpu.VMEM((K,), jnp.int32),
            pltpu.VMEM((K,), jnp.int32),
            pltpu.VMEM((K,), jnp.int32),
            pltpu.VMEM((K,), jnp.int32),
            pltpu.VMEM((K, D), jnp.float32),
            pltpu.VMEM((K, D), jnp.float32),
            pltpu.VMEM((ZR, D), jnp.float32),
            pltpu.VMEM_SHARED((N_PAD, D), jnp.float32),
            pltpu.SemaphoreType.DMA,
            pltpu.SemaphoreType.DMA,
            pltpu.SemaphoreType.DMA,
            pltpu.SemaphoreType.DMA,
            pltpu.SemaphoreType.DMA,
        ],
    )
    def agg(src_hbm, dst_hbm, sup_hbm, out_hbm,
            sidx0, sidx1, didx0, didx1, rb0, rb1, zbuf, acc,
            semi0, semi1, semg0, semg1, semz):
        c = lax.axis_index("c")
        s = lax.axis_index("s")
        w = s * NC + c
        sidx = (sidx0, sidx1)
        didx = (didx0, didx1)
        rb = (rb0, rb1)
        semi = (semi0, semi1)
        semg = (semg0, semg1)

        def istart(i, p):
            pltpu.async_copy(src_hbm.at[w, i], sidx[p], semi[p])
            pltpu.async_copy(dst_hbm.at[w, i], didx[p], semi[p])

        def iwait(i, p):
            pltpu.make_async_copy(src_hbm.at[w, i], sidx[p], semi[p]).wait()
            pltpu.make_async_copy(dst_hbm.at[w, i], didx[p], semi[p]).wait()

        def gstart(p):
            pltpu.async_copy(sup_hbm.at[sidx[p]], rb[p], semg[p])

        def gwait(p):
            pltpu.make_async_copy(sup_hbm.at[sidx[p]], rb[p], semg[p]).wait()

        def scat(p):
            pltpu.sync_copy(rb[p], acc.at[didx[p]], add=True)

        istart(0, 0)
        istart(1, 1)

        zero16 = jnp.zeros((16,), jnp.float32)

        def zrow(r, carry):
            for q in range(D // 16):
                zbuf[r, pl.ds(q * 16, 16)] = zero16
            return carry

        lax.fori_loop(0, ZR, zrow, 0)
        iwait(0, 0)
        gstart(0)
        for t in range(RPS // ZR):
            pltpu.async_copy(zbuf, acc.at[pl.ds(s * RPS + t * ZR, ZR)], semz)
        for t in range(RPS // ZR):
            pltpu.make_async_copy(zbuf, acc.at[pl.ds(s * RPS + t * ZR, ZR)],
                                  semz).wait()
        plsc.subcore_barrier()


        def pair(j, carry):
            a = 2 * j
            iwait(a + 1, 1)
            gstart(1)
            gwait(0)
            scat(0)
            istart(a + 2, 0)
            iwait(a + 2, 0)
            gstart(0)
            gwait(1)
            scat(1)
            istart(a + 3, 1)
            return carry

        lax.fori_loop(0, (CH - 1) // 2, pair, 0)
        gwait(0)
        scat(0)
        iwait(CH, 1)

        plsc.subcore_barrier()
        pltpu.sync_copy(acc.at[pl.ds(s * RPS, RPS)],
                        out_hbm.at[c, pl.ds(s * RPS, RPS)])

    return agg


_agg_cache = {}


def _aggregate(support, src3, dst3):
    D = support.shape[1]
    if D not in _agg_cache:
        _agg_cache[D] = _make_sc_agg(D)
    return _agg_cache[D](src3, dst3, support)


def _mm_body(x_ref, w_ref, o_ref):
    o_ref[...] = jnp.dot(x_ref[...], w_ref[...],
                         preferred_element_type=jnp.float32)


def _matmul(x, w, block_rows=2000):
    n, k = x.shape
    ko, m = w.shape
    grid = (n // block_rows,)
    return pl.pallas_call(
        _mm_body,
        grid=grid,
        in_specs=[
            pl.BlockSpec((block_rows, k), lambda i: (i, 0)),
            pl.BlockSpec((ko, m), lambda i: (0, 0)),
        ],
        out_specs=pl.BlockSpec((block_rows, m), lambda i: (i, 0)),
        out_shape=jax.ShapeDtypeStruct((n, m), jnp.float32),
    )(x, w)


def _relu_mm_body(p_ref, b_ref, w_ref, o_ref):
    h = jnp.maximum(p_ref[0] + p_ref[1] + b_ref[...], 0.0)
    o_ref[...] = jnp.dot(h, w_ref[...], preferred_element_type=jnp.float32)


def _relu_matmul(p, b, w, block_rows=2000):
    _, _, k = p.shape
    n = N
    ko, m = w.shape
    b2 = b.reshape(1, k)
    grid = (n // block_rows,)
    return pl.pallas_call(
        _relu_mm_body,
        grid=grid,
        in_specs=[
            pl.BlockSpec((NC, block_rows, k), lambda i: (0, i, 0)),
            pl.BlockSpec((1, k), lambda i: (0, 0)),
            pl.BlockSpec((ko, m), lambda i: (0, 0)),
        ],
        out_specs=pl.BlockSpec((block_rows, m), lambda i: (i, 0)),
        out_shape=jax.ShapeDtypeStruct((n, m), jnp.float32),
    )(p, b2, w)


def _bias_body(p_ref, b_ref, o_ref):
    m = o_ref.shape[-1]
    o_ref[...] = (p_ref[0] + p_ref[1] + b_ref[...])[:, :m]


def _bias_add(p, b, m_out, block_rows=2000):
    _, _, m = p.shape
    n = N
    b2 = b.reshape(1, m)
    grid = (n // block_rows,)
    return pl.pallas_call(
        _bias_body,
        grid=grid,
        in_specs=[
            pl.BlockSpec((NC, block_rows, m), lambda i: (0, i, 0)),
            pl.BlockSpec((1, m), lambda i: (0, 0)),
        ],
        out_specs=pl.BlockSpec((block_rows, m_out), lambda i: (i, 0)),
        out_shape=jax.ShapeDtypeStruct((n, m_out), jnp.float32),
    )(p, b2)


def _dists_body(er_ref, ec_ref, o_ref):
    i = pl.program_id(0)
    j = pl.program_id(1)
    acc = lax.dot_general(er_ref[...], ec_ref[...],
                          (((1,), (1,)), ((), ())),
                          preferred_element_type=jnp.float32)

    @pl.when(i == j)
    def _():
        ri = lax.broadcasted_iota(jnp.int32, acc.shape, 0)
        ci = lax.broadcasted_iota(jnp.int32, acc.shape, 1)
        o_ref[...] = jnp.where(ri == ci, 0.0, acc)

    @pl.when(i != j)
    def _():
        o_ref[...] = acc


def _dists(embeds, block=1024):
    n, d = embeds.shape
    grid = (pl.cdiv(n, block), pl.cdiv(n, block))
    return pl.pallas_call(
        _dists_body,
        grid=grid,
        in_specs=[
            pl.BlockSpec((block, d), lambda i, j: (i, 0)),
            pl.BlockSpec((block, d), lambda i, j: (j, 0)),
        ],
        out_specs=pl.BlockSpec((block, block), lambda i, j: (i, j)),
        out_shape=jax.ShapeDtypeStruct((n, n), jnp.float32),
    )(embeds, embeds)


def kernel(x, edge_index, W1, b1, W2, b2, W3, b3):
    e2 = edge_index.reshape(2, NW, EPW)
    src_p = jnp.pad(e2[0], ((0, 0), (0, EPW_PAD - EPW)))
    pad_dst = jnp.broadcast_to(N + jnp.arange(NW, dtype=jnp.int32)[:, None],
                               (NW, EPW_PAD - EPW)) if EPW_PAD > EPW else None
    dst_p = (jnp.concatenate([e2[1], pad_dst], axis=1)
             if EPW_PAD > EPW else e2[1])
    pad = jnp.zeros((NW, 2, K), jnp.int32)
    src3 = jnp.concatenate([src_p.reshape(NW, CH, K), pad], axis=1)
    dst3 = jnp.concatenate([dst_p.reshape(NW, CH, K), pad], axis=1)

    W3p = jnp.pad(W3, ((0, 0), (0, 128 - W3.shape[1])))
    b3p = jnp.pad(b3, (0, 128 - b3.shape[0]))

    s1 = _matmul(x, W1)
    p1 = _aggregate(s1, src3, dst3)
    s2 = _relu_matmul(p1, b1, W2)
    p2 = _aggregate(s2, src3, dst3)
    s3 = _relu_matmul(p2, b2, W3p)
    p3 = _aggregate(s3, src3, dst3)
    embeds = _bias_add(p3, b3p, W3.shape[1])
    dists = _dists(embeds)
    return (embeds, dists)
